# baseline (speedup 1.0000x reference)
import jax
import jax.numpy as jnp
from jax import lax
from jax.experimental import pallas as pl
from jax.experimental.pallas import tpu as pltpu
from jax.experimental.pallas import tpu_sc as plsc

B = 16384
F = 26
FP = 32
G = FP // 8
D = 16
N_ROWS = B * FP
NC, NS = 2, 16
NW = NC * NS
ROWS_PER_W = N_ROWS // NW
CHUNK = 1024
SUB = 128
N_CHUNKS = ROWS_PER_W // CHUNK
N_SUB = CHUNK // SUB

BM = 512


def _sc_gather_body(idx_hbm, table_hbm, out_hbm, idx_v, rows_v, sem):
    c = lax.axis_index("c")
    s = lax.axis_index("s")
    wid = s * NC + c
    base = wid * ROWS_PER_W
    pltpu.sync_copy(idx_hbm.at[pl.ds(base, ROWS_PER_W)], idx_v)

    def chunk_body(ci, carry):
        row0 = ci * CHUNK
        copies = []
        for j in range(N_SUB):
            cp = pltpu.make_async_copy(
                table_hbm.at[idx_v.at[pl.ds(row0 + j * SUB, SUB)]],
                rows_v.at[pl.ds(j * SUB, SUB)],
                sem,
            )
            cp.start()
            copies.append(cp)
        for cp in copies:
            cp.wait()
        pltpu.sync_copy(rows_v, out_hbm.at[pl.ds(base + row0, CHUNK)])
        return carry

    lax.fori_loop(0, N_CHUNKS, chunk_body, 0)


_gather = pl.kernel(
    _sc_gather_body,
    out_type=jax.ShapeDtypeStruct((N_ROWS, D), jnp.float32),
    mesh=plsc.VectorSubcoreMesh(core_axis_name="c", subcore_axis_name="s"),
    compiler_params=pltpu.CompilerParams(use_tc_tiling_on_sc=False),
    scratch_types=[
        pltpu.VMEM((ROWS_PER_W,), jnp.int32),
        pltpu.VMEM((CHUNK, D), jnp.float32),
        pltpu.SemaphoreType.DMA,
    ],
)


def _mlp_body(x4, xn, w1p, w1n, b1, w2, b2, w3, b3, o):
    h = jnp.dot(x4[0], w1p[0], preferred_element_type=jnp.float32)
    for j in range(1, G):
        h = h + jnp.dot(x4[j], w1p[j], preferred_element_type=jnp.float32)
    h = h + jnp.dot(xn[...], w1n[...], preferred_element_type=jnp.float32)
    h = jnp.maximum(h + b1[...], 0.0)
    h = jnp.maximum(jnp.dot(h, w2[...], preferred_element_type=jnp.float32) + b2[...], 0.0)
    o[...] = jnp.dot(h, w3[...], preferred_element_type=jnp.float32) + b3[...]


def _mlp(x4, xn, w1p, w1n, b1, w2, b2, w3, b3):
    nn = xn.shape[1]
    h1 = w2.shape[0]
    h2 = w2.shape[1]
    return pl.pallas_call(
        _mlp_body,
        grid=(B // BM,),
        in_specs=[
            pl.BlockSpec((G, BM, 128), lambda i: (0, i, 0)),
            pl.BlockSpec((BM, nn), lambda i: (i, 0)),
            pl.BlockSpec((G, 128, h1), lambda i: (0, 0, 0)),
            pl.BlockSpec((nn, h1), lambda i: (0, 0)),
            pl.BlockSpec((1, h1), lambda i: (0, 0)),
            pl.BlockSpec((h1, h2), lambda i: (0, 0)),
            pl.BlockSpec((1, h2), lambda i: (0, 0)),
            pl.BlockSpec((h2, 1), lambda i: (0, 0)),
            pl.BlockSpec((1, 1), lambda i: (0, 0)),
        ],
        out_specs=pl.BlockSpec((BM, 1), lambda i: (i, 0)),
        out_shape=jax.ShapeDtypeStruct((B, 1), jnp.float32),
    )(x4, xn, w1p, w1n, b1, w2, b2, w3, b3)


def kernel(x_categorical, x_numerical, emb_table, W1, b1, W2, b2, W3, b3):
    xc = x_categorical.astype(jnp.int32)
    xcp = jnp.pad(xc, ((0, 0), (0, FP - F)))
    idxr = xcp.T.reshape(G, 8, B).transpose(0, 2, 1).reshape(-1)
    rows = _gather(idxr, emb_table)
    x4 = rows.reshape(G, B, 128)
    w1p = jnp.pad(W1[: F * D], ((0, G * 128 - F * D), (0, 0))).reshape(G, 128, -1)
    out = _mlp(
        x4,
        x_numerical,
        w1p,
        W1[F * D :],
        b1.reshape(1, -1),
        W2,
        b2.reshape(1, -1),
        W3,
        b3.reshape(1, -1),
    )
    return out.reshape(B)


# speedup vs baseline: 1.3535x; 1.3535x over previous
import jax
import jax.numpy as jnp
from jax import lax
from jax.experimental import pallas as pl
from jax.experimental.pallas import tpu as pltpu
from jax.experimental.pallas import tpu_sc as plsc

B = 16384
F = 26
FP = 32
G = FP // 8
D = 16
N_ROWS = B * FP
NC, NS = 2, 16
NW = NC * NS
ROWS_PER_W = N_ROWS // NW
CHUNK = 1024
SUB = 128
N_CHUNKS = ROWS_PER_W // CHUNK
N_SUB = CHUNK // SUB

BM = 512


def _sc_gather_body(idx_hbm, table_hbm, out_hbm, idx_v, rows_v, sem):
    c = lax.axis_index("c")
    s = lax.axis_index("s")
    wid = s * NC + c
    base = wid * ROWS_PER_W
    pltpu.sync_copy(idx_hbm.at[pl.ds(base, ROWS_PER_W)], idx_v)

    def chunk_body(ci, carry):
        row0 = ci * CHUNK
        copies = []
        for j in range(N_SUB):
            cp = pltpu.make_async_copy(
                table_hbm.at[idx_v.at[pl.ds(row0 + j * SUB, SUB)]],
                rows_v.at[pl.ds(j * SUB, SUB)],
                sem,
            )
            cp.start()
            copies.append(cp)
        for cp in copies:
            cp.wait()
        pltpu.sync_copy(rows_v, out_hbm.at[pl.ds(base + row0, CHUNK)])
        return carry

    lax.fori_loop(0, N_CHUNKS, chunk_body, 0)


_gather = pl.kernel(
    _sc_gather_body,
    out_type=jax.ShapeDtypeStruct((N_ROWS, D), jnp.float32),
    mesh=plsc.VectorSubcoreMesh(core_axis_name="c", subcore_axis_name="s"),
    compiler_params=pltpu.CompilerParams(use_tc_tiling_on_sc=False),
    scratch_types=[
        pltpu.VMEM((ROWS_PER_W,), jnp.int32),
        pltpu.VMEM((CHUNK, D), jnp.float32),
        pltpu.SemaphoreType.DMA,
    ],
)


BK = 4096


def _transpose_body(xt, eye, o):
    o[...] = jax.lax.dot_general(
        xt[...], eye[...], (((0,), (0,)), ((), ())),
        preferred_element_type=jnp.float32,
    )


def _relayout_table(table_t):
    v = table_t.shape[1]
    grid = (v + BK - 1) // BK
    eye = jnp.eye(16, dtype=jnp.float32)
    return pl.pallas_call(
        _transpose_body,
        grid=(grid,),
        in_specs=[
            pl.BlockSpec((16, BK), lambda i: (0, i)),
            pl.BlockSpec((16, 16), lambda i: (0, 0)),
        ],
        out_specs=pl.BlockSpec((BK, 16), lambda i: (i, 0)),
        out_shape=jax.ShapeDtypeStruct((v, 16), jnp.float32),
    )(table_t, eye)


def _mlp_body(x4, xn, w1p, w1n, b1, w2, b2, w3, b3, o):
    h = jnp.dot(x4[0], w1p[0], preferred_element_type=jnp.float32)
    for j in range(1, G):
        h = h + jnp.dot(x4[j], w1p[j], preferred_element_type=jnp.float32)
    h = h + jnp.dot(xn[...], w1n[...], preferred_element_type=jnp.float32)
    h = jnp.maximum(h + b1[...], 0.0)
    h = jnp.maximum(jnp.dot(h, w2[...], preferred_element_type=jnp.float32) + b2[...], 0.0)
    o[...] = jnp.dot(h, w3[...], preferred_element_type=jnp.float32) + b3[...]


def _mlp(x4, xn, w1p, w1n, b1, w2, b2, w3, b3):
    nn = xn.shape[1]
    h1 = w2.shape[0]
    h2 = w2.shape[1]
    return pl.pallas_call(
        _mlp_body,
        grid=(B // BM,),
        in_specs=[
            pl.BlockSpec((G, BM, 128), lambda i: (0, i, 0)),
            pl.BlockSpec((BM, nn), lambda i: (i, 0)),
            pl.BlockSpec((G, 128, h1), lambda i: (0, 0, 0)),
            pl.BlockSpec((nn, h1), lambda i: (0, 0)),
            pl.BlockSpec((1, h1), lambda i: (0, 0)),
            pl.BlockSpec((h1, h2), lambda i: (0, 0)),
            pl.BlockSpec((1, h2), lambda i: (0, 0)),
            pl.BlockSpec((h2, 1), lambda i: (0, 0)),
            pl.BlockSpec((1, 1), lambda i: (0, 0)),
        ],
        out_specs=pl.BlockSpec((BM, 1), lambda i: (i, 0)),
        out_shape=jax.ShapeDtypeStruct((B, 1), jnp.float32),
    )(x4, xn, w1p, w1n, b1, w2, b2, w3, b3)


def kernel(x_categorical, x_numerical, emb_table, W1, b1, W2, b2, W3, b3):
    xc = x_categorical.astype(jnp.int32)
    xcp = jnp.concatenate([xc, xc[:, : FP - F]], axis=1)
    idxr = xcp.T.reshape(G, 8, B).transpose(0, 2, 1).reshape(-1)
    table_rm = _relayout_table(emb_table.T)
    rows = _gather(idxr, table_rm)
    x4 = rows.reshape(G, B, 128)
    w1p = jnp.pad(W1[: F * D], ((0, G * 128 - F * D), (0, 0))).reshape(G, 128, -1)
    out = _mlp(
        x4,
        x_numerical,
        w1p,
        W1[F * D :],
        b1.reshape(1, -1),
        W2,
        b2.reshape(1, -1),
        W3,
        b3.reshape(1, -1),
    )
    return out.reshape(B)


# speedup vs baseline: 2.3411x; 1.7296x over previous
import jax
import jax.numpy as jnp
from jax import lax
from jax.experimental import pallas as pl
from jax.experimental.pallas import tpu as pltpu
from jax.experimental.pallas import tpu_sc as plsc

B = 16384
F = 26
FP = 32
G = FP // 8
D = 16
NC, NS = 2, 16
NW = NC * NS
CHUNK = 1024
SUB = 128
N_SUB = CHUNK // SUB

NCH = 4
BCH = B // NCH
N_ROWS_CH = BCH * FP

BM = 512
BK = 4096


def _make_gather(n_rows):
    rows_per_w = n_rows // NW
    n_chunks = rows_per_w // CHUNK

    def body(idx_hbm, table_hbm, out_hbm, idx_v, rows_v, sem):
        c = lax.axis_index("c")
        s = lax.axis_index("s")
        wid = s * NC + c
        base = wid * rows_per_w
        pltpu.sync_copy(idx_hbm.at[pl.ds(base, rows_per_w)], idx_v)

        def chunk_body(ci, carry):
            row0 = ci * CHUNK
            copies = []
            for j in range(N_SUB):
                cp = pltpu.make_async_copy(
                    table_hbm.at[idx_v.at[pl.ds(row0 + j * SUB, SUB)]],
                    rows_v.at[pl.ds(j * SUB, SUB)],
                    sem,
                )
                cp.start()
                copies.append(cp)
            for cp in copies:
                cp.wait()
            pltpu.sync_copy(rows_v, out_hbm.at[pl.ds(base + row0, CHUNK)])
            return carry

        lax.fori_loop(0, n_chunks, chunk_body, 0)

    return pl.kernel(
        body,
        out_type=jax.ShapeDtypeStruct((n_rows, D), jnp.float32),
        mesh=plsc.VectorSubcoreMesh(core_axis_name="c", subcore_axis_name="s"),
        compiler_params=pltpu.CompilerParams(use_tc_tiling_on_sc=False),
        scratch_types=[
            pltpu.VMEM((rows_per_w,), jnp.int32),
            pltpu.VMEM((CHUNK, D), jnp.float32),
            pltpu.SemaphoreType.DMA,
        ],
    )


_gather_ch = _make_gather(N_ROWS_CH)


def _transpose_body(xt, o):
    x = xt[...]
    s = BK // 8
    pieces = [x[:, k * s:(k + 1) * s].T for k in range(8)]
    o[...] = jnp.concatenate(pieces, axis=1)


def _relayout_table(table_t):
    v = table_t.shape[1]
    grid = (v + BK - 1) // BK
    vp8 = (v + 7) // 8 + (-((v + 7) // 8)) % (BK // 8)
    return pl.pallas_call(
        _transpose_body,
        grid=(grid,),
        in_specs=[
            pl.BlockSpec((16, BK), lambda i: (0, i)),
        ],
        out_specs=pl.BlockSpec((BK // 8, 128), lambda i: (i, 0)),
        out_shape=jax.ShapeDtypeStruct((vp8, 128), jnp.float32),
    )(table_t)


def _mlp_body(x4, xn, w1p, w1n, b1, w2, b2, w3, b3, o):
    h = jnp.dot(x4[0], w1p[0], preferred_element_type=jnp.float32)
    for j in range(1, G):
        h = h + jnp.dot(x4[j], w1p[j], preferred_element_type=jnp.float32)
    h = h + jnp.dot(xn[...], w1n[...], preferred_element_type=jnp.float32)
    h = jnp.maximum(h + b1[...], 0.0)
    h = jnp.maximum(jnp.dot(h, w2[...], preferred_element_type=jnp.float32) + b2[...], 0.0)
    o[...] = jnp.dot(h, w3[...], preferred_element_type=jnp.float32) + b3[...]


def _mlp(x4, xn, w1p, w1n, b1, w2, b2, w3, b3):
    nn = xn.shape[1]
    h1 = w2.shape[0]
    h2 = w2.shape[1]
    return pl.pallas_call(
        _mlp_body,
        grid=(BCH // BM,),
        in_specs=[
            pl.BlockSpec((G, BM, 128), lambda i: (0, i, 0)),
            pl.BlockSpec((BM, nn), lambda i: (i, 0)),
            pl.BlockSpec((G, 128, h1), lambda i: (0, 0, 0)),
            pl.BlockSpec((nn, h1), lambda i: (0, 0)),
            pl.BlockSpec((1, h1), lambda i: (0, 0)),
            pl.BlockSpec((h1, h2), lambda i: (0, 0)),
            pl.BlockSpec((1, h2), lambda i: (0, 0)),
            pl.BlockSpec((h2, 1), lambda i: (0, 0)),
            pl.BlockSpec((1, 1), lambda i: (0, 0)),
        ],
        out_specs=pl.BlockSpec((BM, 1), lambda i: (i, 0)),
        out_shape=jax.ShapeDtypeStruct((BCH, 1), jnp.float32),
    )(x4, xn, w1p, w1n, b1, w2, b2, w3, b3)


def kernel(x_categorical, x_numerical, emb_table, W1, b1, W2, b2, W3, b3):
    xc = x_categorical.astype(jnp.int32)
    xcp = jnp.concatenate([xc, xc[:, : FP - F]], axis=1)
    idxr = (xcp.T.reshape(G, 8, NCH, BCH)
            .transpose(2, 0, 3, 1)
            .reshape(NCH, G * BCH * 8))
    a = idxr % BK
    pidx = (idxr - a) + (a % (BK // 8)) * 8 + a // (BK // 8)
    tpad = _relayout_table(emb_table.T)
    table_rm = tpad.reshape(tpad.shape[0] * 8, 16)
    w1p = jnp.pad(W1[: F * D], ((0, G * 128 - F * D), (0, 0))).reshape(G, 128, -1)
    w1n = W1[F * D:]
    b1r = b1.reshape(1, -1)
    b2r = b2.reshape(1, -1)
    b3r = b3.reshape(1, -1)
    outs = []
    for ch in range(NCH):
        rows = _gather_ch(pidx[ch], table_rm)
        x4 = rows.reshape(G, BCH, 128)
        xn = lax.slice_in_dim(x_numerical, ch * BCH, (ch + 1) * BCH, axis=0)
        outs.append(_mlp(x4, xn, w1p, w1n, b1r, W2, b2r, W3, b3r))
    return jnp.concatenate(outs, axis=0).reshape(B)
